# baseline (speedup 1.0000x reference)
import jax
import jax.numpy as jnp
from jax.experimental import pallas as pl
from jax.experimental.pallas import tpu as pltpu

_S = 0.1
_R2 = 0.02
_EPS = 1e-12


def _scalar_chain(t, zz, n):
    beta1 = (jnp.clip(t, -_S, _S) - t) * (1.0 / n)
    ok1 = zz + (2.0 * t + n * beta1) * beta1 <= _R2
    scale = jnp.minimum(1.0, jnp.sqrt(_R2) * jax.lax.rsqrt(jnp.maximum(zz, _EPS)))
    ok2 = jnp.abs(t) * scale <= _S
    denom = jnp.maximum(n * zz - t * t, _EPS)
    c = jnp.sqrt(jnp.maximum(n * _R2 - _S * _S, 0.0)) * jax.lax.rsqrt(denom)
    beta3 = (jnp.sign(t) * _S - c * t) * (1.0 / n)
    alpha = jnp.where(ok1, 1.0, jnp.where(ok2, scale, c))
    beta = jnp.where(ok1, beta1, jnp.where(ok2, 0.0, beta3))
    return alpha, beta


def _half(x, w, b):
    zt = jax.lax.dot_general(
        w, x,
        dimension_numbers=(((1,), (1,)), ((), ())),
        preferred_element_type=jnp.float32,
    )
    zt = zt - b
    n = zt.shape[0]
    t = jnp.sum(zt, axis=0, keepdims=True)
    zz = jnp.sum(zt * zt, axis=0, keepdims=True)
    alpha, beta = _scalar_chain(t, zz, n)
    return (alpha * zt + beta).T


def _body(x1_ref, x2_ref, w_ref, b_ref, o_ref):
    o_ref[0] = _half(x1_ref[0], w_ref[...], b_ref[...])
    o_ref[1] = _half(x2_ref[0], w_ref[...], b_ref[...])


def kernel(x, W, b):
    B, D_in = x.shape
    D_out = W.shape[0]
    H = 4096
    NB = B // H
    b2 = b.reshape(D_out, 1)
    x3 = x.reshape(NB, H, D_in)
    o3 = pl.pallas_call(
        _body,
        grid=(NB // 2,),
        in_specs=[
            pl.BlockSpec((1, H, D_in), lambda i: (2 * i, 0, 0)),
            pl.BlockSpec((1, H, D_in), lambda i: (2 * i + 1, 0, 0)),
            pl.BlockSpec((D_out, D_in), lambda i: (0, 0)),
            pl.BlockSpec((D_out, 1), lambda i: (0, 0)),
        ],
        out_specs=pl.BlockSpec((2, H, D_out), lambda i: (i, 0, 0)),
        out_shape=jax.ShapeDtypeStruct((NB, H, D_out), jnp.float32),
        compiler_params=pltpu.CompilerParams(
            dimension_semantics=("arbitrary",),
        ),
        name="optlayer_fused",
    )(x3, x3, W, b2)
    return o3.reshape(B, D_out)
